# baseline (speedup 1.0000x reference)
import functools

import numpy as np
import jax
import jax.numpy as jnp
from jax import lax
from jax.experimental import pallas as pl
from jax.experimental.pallas import tpu as pltpu
from jax.experimental.pallas import tpu_sc as plsc

B = 4
H = 128
W = 128
N = H * W
C = 256
NUM_CLASSES = 91
K = 900
KPAD = 1024
CPAD = 128
WH_RATIO_CLIP = 0.016
MAX_RATIO = float(np.abs(np.log(WH_RATIO_CLIP)))

TBLK = 2048
RBLK = 512

NV = 16
VSZ = 1024


def _scores_kernel(mem_ref, wp_ref, bp_ref, out_ref):
    blk = pl.program_id(1)
    mem = mem_ref[0]
    logits = jnp.dot(mem, wp_ref[...],
                     preferred_element_type=jnp.float32)
    logits = logits + bp_ref[...]
    scores = jnp.max(jnp.transpose(logits), axis=0, keepdims=True)
    tok = blk * TBLK + lax.broadcasted_iota(jnp.int32, (1, TBLK), 1)
    x = tok & (W - 1)
    y = (tok >> 7) & (H - 1)
    valid = (x > 0) & (x < W - 1) & (y > 0) & (y < H - 1)
    out_ref[...] = jnp.where(valid, scores, -jnp.inf)[None]


def _scores_call(memory, wp, bp_t):
    return pl.pallas_call(
        _scores_kernel,
        grid=(B, N // TBLK),
        in_specs=[
            pl.BlockSpec((1, TBLK, C), lambda b, i: (b, i, 0)),
            pl.BlockSpec((C, CPAD), lambda b, i: (0, 0)),
            pl.BlockSpec((1, CPAD), lambda b, i: (0, 0)),
        ],
        out_specs=pl.BlockSpec((1, 1, TBLK), lambda b, i: (b, 0, i)),
        out_shape=jax.ShapeDtypeStruct((B, 1, N), jnp.float32),
    )(memory, wp, bp_t)


def _beats(ka, ia, kb, ib):
    return (ka > kb) | ((ka == kb) & (ia < ib))


def _topk_kernel(s_ref, idx_ref):
    keys = [s_ref[0, v * 8:(v + 1) * 8, :] for v in range(NV)]
    sub = lax.broadcasted_iota(jnp.int32, (8, 128), 0)
    lane = lax.broadcasted_iota(jnp.int32, (8, 128), 1)
    pos = sub * 128 + lane
    idxs = [v * VSZ + pos for v in range(NV)]

    for m in range(1, 15):
        k = 1 << m
        for jj in range(m - 1, -1, -1):
            j = 1 << jj
            if j >= VSZ:
                jv = j >> 10
                for v in range(NV):
                    p = v ^ jv
                    if p <= v:
                        continue
                    a_wins = _beats(keys[v], idxs[v], keys[p], idxs[p])
                    winner_low = ((v * VSZ) & k) == 0
                    wk = jnp.where(a_wins, keys[v], keys[p])
                    wi = jnp.where(a_wins, idxs[v], idxs[p])
                    lk = jnp.where(a_wins, keys[p], keys[v])
                    li = jnp.where(a_wins, idxs[p], idxs[v])
                    if winner_low:
                        keys[v], idxs[v], keys[p], idxs[p] = wk, wi, lk, li
                    else:
                        keys[v], idxs[v], keys[p], idxs[p] = lk, li, wk, wi
            else:
                if j < 128:
                    axis, shift, size = 1, j, 128
                else:
                    axis, shift, size = 0, j >> 7, 8
                bitset = ((pos >> jj) & 1) == 1
                if k >= VSZ:
                    dir_elem = None
                else:
                    dir_elem = (pos & k) != 0
                for v in range(NV):
                    pk = jnp.where(bitset,
                                   pltpu.roll(keys[v], shift, axis),
                                   pltpu.roll(keys[v], size - shift, axis))
                    pi = jnp.where(bitset,
                                   pltpu.roll(idxs[v], shift, axis),
                                   pltpu.roll(idxs[v], size - shift, axis))
                    self_wins = _beats(keys[v], idxs[v], pk, pi)
                    if dir_elem is None:
                        dirbit = ((v * VSZ) & k) != 0
                        hold_winner = (bitset == dirbit)
                    else:
                        hold_winner = ~(bitset ^ dir_elem)
                    keep = ~(self_wins ^ hold_winner)
                    keys[v] = jnp.where(keep, keys[v], pk)
                    idxs[v] = jnp.where(keep, idxs[v], pi)

    idx_ref[...] = idxs[0].reshape(1, 8, 128)


def _topk_call(scores3):
    return pl.pallas_call(
        _topk_kernel,
        grid=(B,),
        in_specs=[pl.BlockSpec((1, H, W), lambda b: (b, 0, 0))],
        out_specs=pl.BlockSpec((1, 8, 128), lambda b: (b, 0, 0)),
        out_shape=jax.ShapeDtypeStruct((B, 8, 128), jnp.int32),
    )(scores3)


def _sc_gather(table, idx):
    info = plsc.get_sparse_core_info()
    ncores, nsub = info.num_cores, info.num_subcores
    nw = ncores * nsub
    btot = idx.shape[0]
    b_per_w = btot // nw
    mesh = plsc.VectorSubcoreMesh(core_axis_name="c", subcore_axis_name="s")

    @functools.partial(
        pl.kernel, mesh=mesh,
        out_type=jax.ShapeDtypeStruct((btot, C), jnp.float32),
        scratch_types=[
            pltpu.VMEM((b_per_w,), jnp.int32),
            pltpu.VMEM((b_per_w, C), jnp.float32),
            pltpu.SemaphoreType.DMA,
        ],
    )
    def gather_k(table_hbm, idx_hbm, out_hbm, idx_v, rows_v, sem):
        wid = lax.axis_index("s") * ncores + lax.axis_index("c")
        base = wid * b_per_w
        pltpu.sync_copy(idx_hbm.at[pl.ds(base, b_per_w)], idx_v)
        pltpu.async_copy(table_hbm.at[idx_v], rows_v, sem).wait()
        pltpu.sync_copy(rows_v, out_hbm.at[pl.ds(base, b_per_w)])

    return gather_k(table, idx)


def _mlp_kernel(f_ref, i_ref, w1_ref, b1_ref, w2_ref, b2_ref,
                w3_ref, b3_ref, wc_ref, bc_ref, box_ref, lg_ref):
    x = f_ref[...]
    h = jnp.maximum(jnp.dot(x, w1_ref[...],
                            preferred_element_type=jnp.float32)
                    + b1_ref[...], 0.0)
    h = jnp.maximum(jnp.dot(h, w2_ref[...],
                            preferred_element_type=jnp.float32)
                    + b2_ref[...], 0.0)
    delta = jnp.dot(h, w3_ref[...],
                    preferred_element_type=jnp.float32) + b3_ref[...]
    lg_ref[...] = jnp.dot(x, wc_ref[...],
                          preferred_element_type=jnp.float32) + bc_ref[...]

    t = i_ref[...]
    xq = (t & (W - 1)).astype(jnp.float32)
    yq = ((t >> 7) & (H - 1)).astype(jnp.float32)
    cx = (xq + 0.5) * (1.0 / W)
    cy = (yq + 0.5) * (1.0 / H)
    dx = delta[:, 0:1]
    dy = delta[:, 1:2]
    dw = jnp.clip(delta[:, 2:3], -MAX_RATIO, MAX_RATIO)
    dh = jnp.clip(delta[:, 3:4], -MAX_RATIO, MAX_RATIO)
    gx = cx + 0.05 * dx
    gy = cy + 0.05 * dy
    gw = 0.05 * jnp.exp(dw)
    gh = 0.05 * jnp.exp(dh)
    x1 = jnp.clip(gx - 0.5 * gw, 0.0, 1.0)
    y1 = jnp.clip(gy - 0.5 * gh, 0.0, 1.0)
    x2 = jnp.clip(gx + 0.5 * gw, 0.0, 1.0)
    y2 = jnp.clip(gy + 0.5 * gh, 0.0, 1.0)
    lane = lax.broadcasted_iota(jnp.int32, (RBLK, CPAD), 1)
    box_ref[...] = jnp.where(lane == 0, x1,
                    jnp.where(lane == 1, y1,
                     jnp.where(lane == 2, x2, y2)))


def _mlp_call(feats, idx_col, w1, b1, w2, b2, w3p, b3p, wcp, bcp):
    btot = feats.shape[0]
    full = lambda r, c: pl.BlockSpec((r, c), lambda i: (0, 0))
    return pl.pallas_call(
        _mlp_kernel,
        grid=(btot // RBLK,),
        in_specs=[
            pl.BlockSpec((RBLK, C), lambda i: (i, 0)),
            pl.BlockSpec((RBLK, 1), lambda i: (i, 0)),
            full(C, C), full(1, C), full(C, C), full(1, C),
            full(C, CPAD), full(1, CPAD), full(C, CPAD), full(1, CPAD),
        ],
        out_specs=[
            pl.BlockSpec((RBLK, CPAD), lambda i: (i, 0)),
            pl.BlockSpec((RBLK, CPAD), lambda i: (i, 0)),
        ],
        out_shape=[
            jax.ShapeDtypeStruct((btot, CPAD), jnp.float32),
            jax.ShapeDtypeStruct((btot, CPAD), jnp.float32),
        ],
    )(feats, idx_col, w1, b1, w2, b2, w3p, b3p, wcp, bcp)


def kernel(memory, W_cls, b_cls, W1, b1, W2, b2, W3, b3):
    f32 = jnp.float32
    pad_c = CPAD - NUM_CLASSES
    wcp = jnp.pad(W_cls, ((0, 0), (0, pad_c)))
    bp_t = jnp.concatenate(
        [b_cls, jnp.full((pad_c,), -jnp.inf, f32)]).reshape(1, CPAD)
    bcp = jnp.pad(b_cls, (0, pad_c)).reshape(1, CPAD)
    w3p = jnp.pad(W3, ((0, 0), (0, CPAD - 4)))
    b3p = jnp.pad(b3, (0, CPAD - 4)).reshape(1, CPAD)
    b1r = b1.reshape(1, C)
    b2r = b2.reshape(1, C)

    scores = _scores_call(memory, wcp, bp_t)
    top_idx = _topk_call(scores.reshape(B, H, W))
    top_idx = top_idx.reshape(B, KPAD)
    gidx = (top_idx + jnp.arange(B, dtype=jnp.int32)[:, None] * N
            ).reshape(B * KPAD)
    feats = jnp.take(memory.reshape(B * N, C), gidx, axis=0)
    boxes_p, logits_p = _mlp_call(
        feats, gidx.reshape(B * KPAD, 1),
        W1, b1r, W2, b2r, w3p, b3p, wcp, bcp)

    feats4 = feats.reshape(B, KPAD, C)[:, :K]
    boxes4 = boxes_p[:, :4].reshape(B, KPAD, 4)[:, :K]
    logits4 = logits_p[:, :NUM_CLASSES].reshape(B, KPAD, NUM_CLASSES)[:, :K]
    return jnp.concatenate([feats4, boxes4, logits4], axis=-1)


# speedup vs baseline: 1.1790x; 1.1790x over previous
import functools

import numpy as np
import jax
import jax.numpy as jnp
from jax import lax
from jax.experimental import pallas as pl
from jax.experimental.pallas import tpu as pltpu
from jax.experimental.pallas import tpu_sc as plsc

B = 4
H = 128
W = 128
N = H * W
C = 256
NUM_CLASSES = 91
K = 900
KPAD = 1024
CPAD = 128
WH_RATIO_CLIP = 0.016
MAX_RATIO = float(np.abs(np.log(WH_RATIO_CLIP)))

TBLK = 2048
RBLK = 512

NV = 16
VSZ = 1024


def _scores_kernel(mem_ref, wp_ref, bp_ref, out_ref):
    blk = pl.program_id(1)
    mem = mem_ref[0]
    logits = jnp.dot(mem, wp_ref[...],
                     preferred_element_type=jnp.float32)
    logits = logits + bp_ref[...]
    scores = jnp.max(jnp.transpose(logits), axis=0, keepdims=True)
    tok = blk * TBLK + lax.broadcasted_iota(jnp.int32, (1, TBLK), 1)
    x = tok & (W - 1)
    y = (tok >> 7) & (H - 1)
    valid = (x > 0) & (x < W - 1) & (y > 0) & (y < H - 1)
    out_ref[...] = jnp.where(valid, scores, -jnp.inf)[None]


def _scores_call(memory, wp, bp_t):
    return pl.pallas_call(
        _scores_kernel,
        grid=(B, N // TBLK),
        in_specs=[
            pl.BlockSpec((1, TBLK, C), lambda b, i: (b, i, 0)),
            pl.BlockSpec((C, CPAD), lambda b, i: (0, 0)),
            pl.BlockSpec((1, CPAD), lambda b, i: (0, 0)),
        ],
        out_specs=pl.BlockSpec((1, 1, TBLK), lambda b, i: (b, 0, i)),
        out_shape=jax.ShapeDtypeStruct((B, 1, N), jnp.float32),
    )(memory, wp, bp_t)


def _beats(ka, ia, kb, ib):
    return (ka > kb) | ((ka == kb) & (ia < ib))


def _topk_kernel(s_ref, idx_ref):
    keys = [s_ref[0, v * 8:(v + 1) * 8, :] for v in range(NV)]
    sub = lax.broadcasted_iota(jnp.int32, (8, 128), 0)
    lane = lax.broadcasted_iota(jnp.int32, (8, 128), 1)
    pos = sub * 128 + lane
    idxs = [v * VSZ + pos for v in range(NV)]

    for m in range(1, 15):
        k = 1 << m
        for jj in range(m - 1, -1, -1):
            j = 1 << jj
            if j >= VSZ:
                jv = j >> 10
                for v in range(NV):
                    p = v ^ jv
                    if p <= v:
                        continue
                    a_wins = _beats(keys[v], idxs[v], keys[p], idxs[p])
                    winner_low = ((v * VSZ) & k) == 0
                    wk = jnp.where(a_wins, keys[v], keys[p])
                    wi = jnp.where(a_wins, idxs[v], idxs[p])
                    lk = jnp.where(a_wins, keys[p], keys[v])
                    li = jnp.where(a_wins, idxs[p], idxs[v])
                    if winner_low:
                        keys[v], idxs[v], keys[p], idxs[p] = wk, wi, lk, li
                    else:
                        keys[v], idxs[v], keys[p], idxs[p] = lk, li, wk, wi
            else:
                if j < 128:
                    axis, shift, size = 1, j, 128
                else:
                    axis, shift, size = 0, j >> 7, 8
                bitset = ((pos >> jj) & 1) == 1
                if k >= VSZ:
                    dir_elem = None
                else:
                    dir_elem = (pos & k) != 0
                for v in range(NV):
                    pk = jnp.where(bitset,
                                   pltpu.roll(keys[v], shift, axis),
                                   pltpu.roll(keys[v], size - shift, axis))
                    pi = jnp.where(bitset,
                                   pltpu.roll(idxs[v], shift, axis),
                                   pltpu.roll(idxs[v], size - shift, axis))
                    self_wins = _beats(keys[v], idxs[v], pk, pi)
                    if dir_elem is None:
                        dirbit = ((v * VSZ) & k) != 0
                        hold_winner = (bitset == dirbit)
                    else:
                        hold_winner = ~(bitset ^ dir_elem)
                    keep = ~(self_wins ^ hold_winner)
                    keys[v] = jnp.where(keep, keys[v], pk)
                    idxs[v] = jnp.where(keep, idxs[v], pi)

    idx_ref[...] = idxs[0].reshape(1, 8, 128)


def _topk_call(scores3):
    return pl.pallas_call(
        _topk_kernel,
        grid=(B,),
        in_specs=[pl.BlockSpec((1, H, W), lambda b: (b, 0, 0))],
        out_specs=pl.BlockSpec((1, 8, 128), lambda b: (b, 0, 0)),
        out_shape=jax.ShapeDtypeStruct((B, 8, 128), jnp.int32),
    )(scores3)


def _sc_gather(table, idx):
    info = plsc.get_sparse_core_info()
    ncores, nsub = info.num_cores, info.num_subcores
    nw = ncores * nsub
    btot = idx.shape[0]
    b_per_w = btot // nw
    mesh = plsc.VectorSubcoreMesh(core_axis_name="c", subcore_axis_name="s")

    @functools.partial(
        pl.kernel, mesh=mesh,
        out_type=jax.ShapeDtypeStruct((btot, C), jnp.float32),
        scratch_types=[
            pltpu.VMEM((b_per_w,), jnp.int32),
            pltpu.VMEM((b_per_w, C), jnp.float32),
            pltpu.SemaphoreType.DMA,
        ],
    )
    def gather_k(table_hbm, idx_hbm, out_hbm, idx_v, rows_v, sem):
        wid = lax.axis_index("s") * ncores + lax.axis_index("c")
        base = wid * b_per_w
        pltpu.sync_copy(idx_hbm.at[pl.ds(base, b_per_w)], idx_v)
        pltpu.async_copy(table_hbm.at[idx_v], rows_v, sem).wait()
        pltpu.sync_copy(rows_v, out_hbm.at[pl.ds(base, b_per_w)])

    return gather_k(table, idx)


def _mlp_kernel(f_ref, i_ref, w1_ref, b1_ref, w2_ref, b2_ref,
                w3_ref, b3_ref, wc_ref, bc_ref, out_ref):
    x = f_ref[...]
    h = jnp.maximum(jnp.dot(x, w1_ref[...],
                            preferred_element_type=jnp.float32)
                    + b1_ref[...], 0.0)
    h = jnp.maximum(jnp.dot(h, w2_ref[...],
                            preferred_element_type=jnp.float32)
                    + b2_ref[...], 0.0)
    delta = jnp.dot(h, w3_ref[...],
                    preferred_element_type=jnp.float32) + b3_ref[...]
    lg = jnp.dot(x, wc_ref[...],
                 preferred_element_type=jnp.float32) + bc_ref[...]

    t = i_ref[...]
    xq = (t & (W - 1)).astype(jnp.float32)
    yq = ((t >> 7) & (H - 1)).astype(jnp.float32)
    cx = (xq + 0.5) * (1.0 / W)
    cy = (yq + 0.5) * (1.0 / H)
    dx = delta[:, 0:1]
    dy = delta[:, 1:2]
    dw = jnp.clip(delta[:, 2:3], -MAX_RATIO, MAX_RATIO)
    dh = jnp.clip(delta[:, 3:4], -MAX_RATIO, MAX_RATIO)
    gx = cx + 0.05 * dx
    gy = cy + 0.05 * dy
    gw = 0.05 * jnp.exp(dw)
    gh = 0.05 * jnp.exp(dh)
    x1 = jnp.clip(gx - 0.5 * gw, 0.0, 1.0)
    y1 = jnp.clip(gy - 0.5 * gh, 0.0, 1.0)
    x2 = jnp.clip(gx + 0.5 * gw, 0.0, 1.0)
    y2 = jnp.clip(gy + 0.5 * gh, 0.0, 1.0)
    out = jnp.concatenate(
        [x[:K], x1[:K], y1[:K], x2[:K], y2[:K], lg[:K, :NUM_CLASSES]],
        axis=1)
    out_ref[...] = out[None]


def _mlp_call(feats, idx_col, w1, b1, w2, b2, w3p, b3p, wcp, bcp):
    full = lambda r, c: pl.BlockSpec((r, c), lambda i: (0, 0))
    return pl.pallas_call(
        _mlp_kernel,
        grid=(B,),
        in_specs=[
            pl.BlockSpec((KPAD, C), lambda i: (i, 0)),
            pl.BlockSpec((KPAD, 1), lambda i: (i, 0)),
            full(C, C), full(1, C), full(C, C), full(1, C),
            full(C, CPAD), full(1, CPAD), full(C, CPAD), full(1, CPAD),
        ],
        out_specs=pl.BlockSpec((1, K, C + 4 + NUM_CLASSES),
                               lambda i: (i, 0, 0)),
        out_shape=jax.ShapeDtypeStruct((B, K, C + 4 + NUM_CLASSES),
                                       jnp.float32),
    )(feats, idx_col, w1, b1, w2, b2, w3p, b3p, wcp, bcp)


def kernel(memory, W_cls, b_cls, W1, b1, W2, b2, W3, b3):
    f32 = jnp.float32
    pad_c = CPAD - NUM_CLASSES
    wcp = jnp.pad(W_cls, ((0, 0), (0, pad_c)))
    bp_t = jnp.concatenate(
        [b_cls, jnp.full((pad_c,), -jnp.inf, f32)]).reshape(1, CPAD)
    bcp = jnp.pad(b_cls, (0, pad_c)).reshape(1, CPAD)
    w3p = jnp.pad(W3, ((0, 0), (0, CPAD - 4)))
    b3p = jnp.pad(b3, (0, CPAD - 4)).reshape(1, CPAD)
    b1r = b1.reshape(1, C)
    b2r = b2.reshape(1, C)

    scores = _scores_call(memory, wcp, bp_t)
    top_idx = _topk_call(scores.reshape(B, H, W))
    top_idx = top_idx.reshape(B, KPAD)
    gidx = (top_idx + jnp.arange(B, dtype=jnp.int32)[:, None] * N
            ).reshape(B * KPAD)
    feats = _sc_gather(memory.reshape(B * N, C), gidx)
    return _mlp_call(
        feats, gidx.reshape(B * KPAD, 1),
        W1, b1r, W2, b2r, w3p, b3p, wcp, bcp)


# speedup vs baseline: 1.2103x; 1.0265x over previous
import functools

import numpy as np
import jax
import jax.numpy as jnp
from jax import lax
from jax.experimental import pallas as pl
from jax.experimental.pallas import tpu as pltpu
from jax.experimental.pallas import tpu_sc as plsc

B = 4
H = 128
W = 128
N = H * W
C = 256
NUM_CLASSES = 91
K = 900
KPAD = 1024
CPAD = 128
WH_RATIO_CLIP = 0.016
MAX_RATIO = float(np.abs(np.log(WH_RATIO_CLIP)))

TBLK = 2048
RBLK = 512

NV = 16
VSZ = 1024


def _scores_kernel(mem_ref, wp_ref, bp_ref, out_ref):
    blk = pl.program_id(1)
    mem = mem_ref[0]
    logits = jnp.dot(mem, wp_ref[...],
                     preferred_element_type=jnp.float32)
    logits = logits + bp_ref[...]
    scores = jnp.max(jnp.transpose(logits), axis=0, keepdims=True)
    tok = blk * TBLK + lax.broadcasted_iota(jnp.int32, (1, TBLK), 1)
    x = tok & (W - 1)
    y = (tok >> 7) & (H - 1)
    valid = (x > 0) & (x < W - 1) & (y > 0) & (y < H - 1)
    out_ref[...] = jnp.where(valid, scores, -jnp.inf)[None]


def _scores_call(memory, wp, bp_t):
    return pl.pallas_call(
        _scores_kernel,
        grid=(B, N // TBLK),
        in_specs=[
            pl.BlockSpec((1, TBLK, C), lambda b, i: (b, i, 0)),
            pl.BlockSpec((C, CPAD), lambda b, i: (0, 0)),
            pl.BlockSpec((1, CPAD), lambda b, i: (0, 0)),
        ],
        out_specs=pl.BlockSpec((1, 1, TBLK), lambda b, i: (b, 0, i)),
        out_shape=jax.ShapeDtypeStruct((B, 1, N), jnp.float32),
    )(memory, wp, bp_t)


def _beats(ka, ia, kb, ib):
    return (ka > kb) | ((ka == kb) & (ia < ib))


def _topk_kernel(s_ref, idx_ref):
    b = pl.program_id(0)
    sub = lax.broadcasted_iota(jnp.int32, (8, 128), 0)
    lane = lax.broadcasted_iota(jnp.int32, (8, 128), 1)
    pos = sub * 128 + lane
    keys = [s_ref[0, v * 8:(v + 1) * 8, :] for v in range(NV)]
    idxs = [b * N + v * VSZ + pos for v in range(NV)]
    bitsets = [((pos >> jj) & 1) == 1 for jj in range(10)]

    def cmpex(jj, pairs):
        j = 1 << jj
        axis, shift, size = (1, j, 128) if j < 128 else (0, j >> 7, 8)
        bs = bitsets[jj]
        for v, hw in pairs:
            pk = jnp.where(bs, pltpu.roll(keys[v], shift, axis),
                           pltpu.roll(keys[v], size - shift, axis))
            pi = jnp.where(bs, pltpu.roll(idxs[v], shift, axis),
                           pltpu.roll(idxs[v], size - shift, axis))
            sw = _beats(keys[v], idxs[v], pk, pi)
            keep = ~(sw ^ hw)
            keys[v] = jnp.where(keep, keys[v], pk)
            idxs[v] = jnp.where(keep, idxs[v], pi)

    for m in range(1, 11):
        k = 1 << m
        for jj in range(m - 1, -1, -1):
            if k < VSZ:
                base = ~(bitsets[jj] ^ ((pos & k) != 0))
            else:
                base = ~bitsets[jj]
            nbase = ~base
            cmpex(jj, [(v, base if v % 2 == 0 else nbase)
                       for v in range(NV)])

    alive = list(range(NV))
    while len(alive) > 1:
        nxt = []
        for t in range(len(alive) // 2):
            a, c = alive[2 * t], alive[2 * t + 1]
            aw = _beats(keys[a], idxs[a], keys[c], idxs[c])
            keys[a] = jnp.where(aw, keys[a], keys[c])
            idxs[a] = jnp.where(aw, idxs[a], idxs[c])
            nxt.append(a)
        for jj in range(9, -1, -1):
            base = ~bitsets[jj]
            nbase = bitsets[jj]
            cmpex(jj, [(v, base if t % 2 == 0 else nbase)
                       for t, v in enumerate(nxt)])
        alive = nxt

    idx_ref[...] = idxs[alive[0]].reshape(1, 8, 128)


def _topk_call(scores2):
    return pl.pallas_call(
        _topk_kernel,
        grid=(B,),
        in_specs=[pl.BlockSpec((1, H, W), lambda b: (b, 0, 0))],
        out_specs=pl.BlockSpec((1, 8, 128), lambda b: (b, 0, 0)),
        out_shape=jax.ShapeDtypeStruct((B, 8, 128), jnp.int32),
    )(scores2)


def _sc_gather(table, idx):
    info = plsc.get_sparse_core_info()
    ncores, nsub = info.num_cores, info.num_subcores
    nw = ncores * nsub
    btot = idx.shape[0]
    b_per_w = btot // nw
    mesh = plsc.VectorSubcoreMesh(core_axis_name="c", subcore_axis_name="s")

    @functools.partial(
        pl.kernel, mesh=mesh,
        out_type=jax.ShapeDtypeStruct((btot, C), jnp.float32),
        scratch_types=[
            pltpu.VMEM((b_per_w,), jnp.int32),
            pltpu.VMEM((b_per_w, C), jnp.float32),
            pltpu.SemaphoreType.DMA,
        ],
    )
    def gather_k(table_hbm, idx_hbm, out_hbm, idx_v, rows_v, sem):
        wid = lax.axis_index("s") * ncores + lax.axis_index("c")
        base = wid * b_per_w
        pltpu.sync_copy(idx_hbm.at[pl.ds(base, b_per_w)], idx_v)
        pltpu.async_copy(table_hbm.at[idx_v], rows_v, sem).wait()
        pltpu.sync_copy(rows_v, out_hbm.at[pl.ds(base, b_per_w)])

    return gather_k(table, idx)


def _mlp_kernel(f_ref, i_ref, w1_ref, b1_ref, w2_ref, b2_ref,
                w3_ref, b3_ref, wc_ref, bc_ref, out_ref):
    x = f_ref[...]
    h = jnp.maximum(jnp.dot(x, w1_ref[...],
                            preferred_element_type=jnp.float32)
                    + b1_ref[...], 0.0)
    h = jnp.maximum(jnp.dot(h, w2_ref[...],
                            preferred_element_type=jnp.float32)
                    + b2_ref[...], 0.0)
    delta = jnp.dot(h, w3_ref[...],
                    preferred_element_type=jnp.float32) + b3_ref[...]
    lg = jnp.dot(x, wc_ref[...],
                 preferred_element_type=jnp.float32) + bc_ref[...]

    t = i_ref[...]
    xq = (t & (W - 1)).astype(jnp.float32)
    yq = ((t >> 7) & (H - 1)).astype(jnp.float32)
    cx = (xq + 0.5) * (1.0 / W)
    cy = (yq + 0.5) * (1.0 / H)
    dx = delta[:, 0:1]
    dy = delta[:, 1:2]
    dw = jnp.clip(delta[:, 2:3], -MAX_RATIO, MAX_RATIO)
    dh = jnp.clip(delta[:, 3:4], -MAX_RATIO, MAX_RATIO)
    gx = cx + 0.05 * dx
    gy = cy + 0.05 * dy
    gw = 0.05 * jnp.exp(dw)
    gh = 0.05 * jnp.exp(dh)
    x1 = jnp.clip(gx - 0.5 * gw, 0.0, 1.0)
    y1 = jnp.clip(gy - 0.5 * gh, 0.0, 1.0)
    x2 = jnp.clip(gx + 0.5 * gw, 0.0, 1.0)
    y2 = jnp.clip(gy + 0.5 * gh, 0.0, 1.0)
    out = jnp.concatenate(
        [x[:K], x1[:K], y1[:K], x2[:K], y2[:K], lg[:K, :NUM_CLASSES]],
        axis=1)
    out_ref[...] = out[None]


def _mlp_call(feats, idx_col, w1, b1, w2, b2, w3p, b3p, wcp, bcp):
    full = lambda r, c: pl.BlockSpec((r, c), lambda i: (0, 0))
    return pl.pallas_call(
        _mlp_kernel,
        grid=(B,),
        in_specs=[
            pl.BlockSpec((KPAD, C), lambda i: (i, 0)),
            pl.BlockSpec((KPAD, 1), lambda i: (i, 0)),
            full(C, C), full(1, C), full(C, C), full(1, C),
            full(C, CPAD), full(1, CPAD), full(C, CPAD), full(1, CPAD),
        ],
        out_specs=pl.BlockSpec((1, K, C + 4 + NUM_CLASSES),
                               lambda i: (i, 0, 0)),
        out_shape=jax.ShapeDtypeStruct((B, K, C + 4 + NUM_CLASSES),
                                       jnp.float32),
    )(feats, idx_col, w1, b1, w2, b2, w3p, b3p, wcp, bcp)


def kernel(memory, W_cls, b_cls, W1, b1, W2, b2, W3, b3):
    f32 = jnp.float32
    pad_c = CPAD - NUM_CLASSES
    wcp = jnp.pad(W_cls, ((0, 0), (0, pad_c)))
    bp_t = jnp.concatenate(
        [b_cls, jnp.full((pad_c,), -jnp.inf, f32)]).reshape(1, CPAD)
    w3p = jnp.pad(W3, ((0, 0), (0, CPAD - 4)))
    b3p = jnp.pad(b3, (0, CPAD - 4)).reshape(1, CPAD)
    b1r = b1.reshape(1, C)
    b2r = b2.reshape(1, C)

    scores = _scores_call(memory, wcp, bp_t)
    gidx = _topk_call(scores.reshape(B, H, W)).reshape(B * KPAD)
    feats = _sc_gather(memory.reshape(B * N, C), gidx)
    return _mlp_call(
        feats, gidx.reshape(B * KPAD, 1),
        W1, b1r, W2, b2r, w3p, b3p, wcp, bp_t)


# speedup vs baseline: 1.2334x; 1.0191x over previous
import functools

import numpy as np
import jax
import jax.numpy as jnp
from jax import lax
from jax.experimental import pallas as pl
from jax.experimental.pallas import tpu as pltpu
from jax.experimental.pallas import tpu_sc as plsc

B = 4
H = 128
W = 128
N = H * W
C = 256
NUM_CLASSES = 91
K = 900
KPAD = 1024
CPAD = 128
WH_RATIO_CLIP = 0.016
MAX_RATIO = float(np.abs(np.log(WH_RATIO_CLIP)))

TBLK = 2048
RBLK = 512

NV = 16
VSZ = 1024


def _beats(ka, ia, kb, ib):
    return (ka > kb) | ((ka == kb) & (ia < ib))


def _seltop_kernel(mem_ref, wp_ref, bp_ref, idx_ref, scr_ref):
    b = pl.program_id(0)
    blk = pl.program_id(1)
    mem = mem_ref[0]
    logits = jnp.dot(mem, wp_ref[...],
                     preferred_element_type=jnp.float32)
    logits = logits + bp_ref[...]
    scores = jnp.max(jnp.transpose(logits), axis=0, keepdims=True)
    tok = blk * TBLK + lax.broadcasted_iota(jnp.int32, (1, TBLK), 1)
    x = tok & (W - 1)
    y = (tok >> 7) & (H - 1)
    valid = (x > 0) & (x < W - 1) & (y > 0) & (y < H - 1)
    scr_ref[pl.ds(blk, 1), :] = jnp.where(valid, scores, -jnp.inf)

    @pl.when(blk == N // TBLK - 1)
    def _sort():
        _topk_body(b, scr_ref, idx_ref)


def _topk_body(b, s_ref, idx_ref):
    sub = lax.broadcasted_iota(jnp.int32, (8, 128), 0)
    lane = lax.broadcasted_iota(jnp.int32, (8, 128), 1)
    pos = sub * 128 + lane
    keys = [s_ref[:, v * 128:(v + 1) * 128] for v in range(NV)]
    idxs = [b * N + sub * TBLK + v * 128 + lane for v in range(NV)]
    bitsets = [((pos >> jj) & 1) == 1 for jj in range(10)]

    def cmpex(jj, pairs):
        j = 1 << jj
        axis, shift, size = (1, j, 128) if j < 128 else (0, j >> 7, 8)
        bs = bitsets[jj]
        for v, hw in pairs:
            pk = jnp.where(bs, pltpu.roll(keys[v], shift, axis),
                           pltpu.roll(keys[v], size - shift, axis))
            pi = jnp.where(bs, pltpu.roll(idxs[v], shift, axis),
                           pltpu.roll(idxs[v], size - shift, axis))
            sw = _beats(keys[v], idxs[v], pk, pi)
            keep = ~(sw ^ hw)
            keys[v] = jnp.where(keep, keys[v], pk)
            idxs[v] = jnp.where(keep, idxs[v], pi)

    for m in range(1, 11):
        k = 1 << m
        for jj in range(m - 1, -1, -1):
            if k < VSZ:
                base = ~(bitsets[jj] ^ ((pos & k) != 0))
            else:
                base = ~bitsets[jj]
            nbase = ~base
            cmpex(jj, [(v, base if v % 2 == 0 else nbase)
                       for v in range(NV)])

    alive = list(range(NV))
    while len(alive) > 1:
        nxt = []
        for t in range(len(alive) // 2):
            a, c = alive[2 * t], alive[2 * t + 1]
            aw = _beats(keys[a], idxs[a], keys[c], idxs[c])
            keys[a] = jnp.where(aw, keys[a], keys[c])
            idxs[a] = jnp.where(aw, idxs[a], idxs[c])
            nxt.append(a)
        for jj in range(9, -1, -1):
            base = ~bitsets[jj]
            nbase = bitsets[jj]
            cmpex(jj, [(v, base if t % 2 == 0 else nbase)
                       for t, v in enumerate(nxt)])
        alive = nxt

    idx_ref[...] = idxs[alive[0]].reshape(1, 8, 128)


def _seltop_call(memory, wp, bp_t):
    return pl.pallas_call(
        _seltop_kernel,
        grid=(B, N // TBLK),
        in_specs=[
            pl.BlockSpec((1, TBLK, C), lambda b, i: (b, i, 0)),
            pl.BlockSpec((C, CPAD), lambda b, i: (0, 0)),
            pl.BlockSpec((1, CPAD), lambda b, i: (0, 0)),
        ],
        out_specs=pl.BlockSpec((1, 8, 128), lambda b, i: (b, 0, 0)),
        out_shape=jax.ShapeDtypeStruct((B, 8, 128), jnp.int32),
        scratch_shapes=[pltpu.VMEM((N // TBLK, TBLK), jnp.float32)],
    )(memory, wp, bp_t)


def _sc_gather(table, idx):
    info = plsc.get_sparse_core_info()
    ncores, nsub = info.num_cores, info.num_subcores
    nw = ncores * nsub
    nrows = idx.shape[0] * idx.shape[1]
    b_per_w = (nrows * idx.shape[2]) // nw
    mesh = plsc.VectorSubcoreMesh(core_axis_name="c", subcore_axis_name="s")

    @functools.partial(
        pl.kernel, mesh=mesh,
        out_type=jax.ShapeDtypeStruct((nw * b_per_w, C), jnp.float32),
        scratch_types=[
            pltpu.VMEM((b_per_w,), jnp.int32),
            pltpu.VMEM((b_per_w, C), jnp.float32),
            pltpu.SemaphoreType.DMA,
        ],
    )
    def gather_k(table_hbm, idx_hbm, out_hbm, idx_v, rows_v, sem):
        wid = lax.axis_index("s") * ncores + lax.axis_index("c")
        nr = idx_hbm.shape[1]
        pltpu.sync_copy(idx_hbm.at[wid // nr, wid % nr], idx_v)
        pltpu.async_copy(table_hbm.at[idx_v], rows_v, sem).wait()
        pltpu.sync_copy(rows_v, out_hbm.at[pl.ds(wid * b_per_w, b_per_w)])

    return gather_k(table, idx)


def _mlp_kernel(f_ref, i_ref, w1_ref, b1_ref, w2_ref, b2_ref,
                w3_ref, b3_ref, wc_ref, bc_ref, out_ref):
    x = f_ref[...]
    h = jnp.maximum(jnp.dot(x, w1_ref[...],
                            preferred_element_type=jnp.float32)
                    + b1_ref[...], 0.0)
    h = jnp.maximum(jnp.dot(h, w2_ref[...],
                            preferred_element_type=jnp.float32)
                    + b2_ref[...], 0.0)
    delta = jnp.dot(h, w3_ref[...],
                    preferred_element_type=jnp.float32) + b3_ref[...]
    lg = jnp.dot(x, wc_ref[...],
                 preferred_element_type=jnp.float32) + bc_ref[...]

    t = i_ref[...]
    xq = (t & (W - 1)).astype(jnp.float32)
    yq = ((t >> 7) & (H - 1)).astype(jnp.float32)
    cx = (xq + 0.5) * (1.0 / W)
    cy = (yq + 0.5) * (1.0 / H)
    dx = delta[:, 0:1]
    dy = delta[:, 1:2]
    dw = jnp.clip(delta[:, 2:3], -MAX_RATIO, MAX_RATIO)
    dh = jnp.clip(delta[:, 3:4], -MAX_RATIO, MAX_RATIO)
    gx = cx + 0.05 * dx
    gy = cy + 0.05 * dy
    gw = 0.05 * jnp.exp(dw)
    gh = 0.05 * jnp.exp(dh)
    x1 = jnp.clip(gx - 0.5 * gw, 0.0, 1.0)
    y1 = jnp.clip(gy - 0.5 * gh, 0.0, 1.0)
    x2 = jnp.clip(gx + 0.5 * gw, 0.0, 1.0)
    y2 = jnp.clip(gy + 0.5 * gh, 0.0, 1.0)
    out = jnp.concatenate(
        [x[:K], x1[:K], y1[:K], x2[:K], y2[:K], lg[:K, :NUM_CLASSES]],
        axis=1)
    out_ref[...] = out[None]


def _mlp_call(feats, idx_col, w1, b1, w2, b2, w3p, b3p, wcp, bcp):
    full = lambda r, c: pl.BlockSpec((r, c), lambda i: (0, 0))
    return pl.pallas_call(
        _mlp_kernel,
        grid=(B,),
        in_specs=[
            pl.BlockSpec((KPAD, C), lambda i: (i, 0)),
            pl.BlockSpec((KPAD, 1), lambda i: (i, 0)),
            full(C, C), full(1, C), full(C, C), full(1, C),
            full(C, 4), full(1, 4), full(C, CPAD), full(1, CPAD),
        ],
        out_specs=pl.BlockSpec((1, K, C + 4 + NUM_CLASSES),
                               lambda i: (i, 0, 0)),
        out_shape=jax.ShapeDtypeStruct((B, K, C + 4 + NUM_CLASSES),
                                       jnp.float32),
    )(feats, idx_col, w1, b1, w2, b2, w3p, b3p, wcp, bcp)


def kernel(memory, W_cls, b_cls, W1, b1, W2, b2, W3, b3):
    f32 = jnp.float32
    pad_c = CPAD - NUM_CLASSES
    wcp = jnp.pad(W_cls, ((0, 0), (0, pad_c)))
    bp_t = jnp.concatenate(
        [b_cls, jnp.full((pad_c,), -jnp.inf, f32)]).reshape(1, CPAD)
    b1r = b1.reshape(1, C)
    b2r = b2.reshape(1, C)
    b3r = b3.reshape(1, 4)

    gidx3 = _seltop_call(memory, wcp, bp_t)
    feats = _sc_gather(memory.reshape(B * N, C), gidx3)
    return _mlp_call(
        feats, gidx3.reshape(B * KPAD, 1),
        W1, b1r, W2, b2r, W3, b3r, wcp, bp_t)


# speedup vs baseline: 1.2776x; 1.0359x over previous
import functools

import numpy as np
import jax
import jax.numpy as jnp
from jax import lax
from jax.experimental import pallas as pl
from jax.experimental.pallas import tpu as pltpu
from jax.experimental.pallas import tpu_sc as plsc

B = 4
H = 128
W = 128
N = H * W
C = 256
NUM_CLASSES = 91
K = 900
KPAD = 1024
CPAD = 128
WH_RATIO_CLIP = 0.016
MAX_RATIO = float(np.abs(np.log(WH_RATIO_CLIP)))

TBLK = 2048
RBLK = 512

NV = 16
VSZ = 1024


def _beats(ka, ia, kb, ib):
    return (ka > kb) | ((ka == kb) & (ia < ib))


NGRP = 8


def _sort_schedule():
    ops = []
    for m in range(1, 11):
        k = 1 << m
        for jj in range(m - 1, -1, -1):
            spec = ('elem', k) if k < VSZ else ('uni',)
            ops.append((('cx', jj, spec,
                         [(v, v % 2 == 1) for v in range(NV)]), NV))
    alive = list(range(NV))
    while len(alive) > 1:
        nxt = []
        for t in range(len(alive) // 2):
            a, c = alive[2 * t], alive[2 * t + 1]
            ops.append((('win', a, c, None), 1))
            nxt.append(a)
        for jj in range(9, -1, -1):
            ops.append((('cx', jj, ('uni',),
                         [(v, t % 2 == 1) for t, v in enumerate(nxt)]),
                        len(nxt)))
        alive = nxt
    total = sum(w for _, w in ops)
    groups, acc, gi = [[] for _ in range(NGRP)], 0, 0
    for op, w in ops:
        if acc >= (gi + 1) * total / NGRP and gi < NGRP - 1:
            gi += 1
        groups[gi].append(op)
        acc += w
    return groups, alive[0]


_SORT_GROUPS, _SORT_ROOT = _sort_schedule()


def _run_sort_ops(opsg, keys, idxs, pos, bitsets):
    for op in opsg:
        if op[0] == 'win':
            _, a, c, _ = op
            aw = _beats(keys[a], idxs[a], keys[c], idxs[c])
            keys[a] = jnp.where(aw, keys[a], keys[c])
            idxs[a] = jnp.where(aw, idxs[a], idxs[c])
            continue
        _, jj, spec, pairs = op
        j = 1 << jj
        axis, shift, size = (1, j, 128) if j < 128 else (0, j >> 7, 8)
        bs = bitsets[jj]
        if spec[0] == 'elem':
            base = ~(bs ^ ((pos & spec[1]) != 0))
        else:
            base = ~bs
        nbase = ~base
        for v, inv in pairs:
            hw = nbase if inv else base
            pk = jnp.where(bs, pltpu.roll(keys[v], shift, axis),
                           pltpu.roll(keys[v], size - shift, axis))
            pi = jnp.where(bs, pltpu.roll(idxs[v], shift, axis),
                           pltpu.roll(idxs[v], size - shift, axis))
            sw = _beats(keys[v], idxs[v], pk, pi)
            keep = ~(sw ^ hw)
            keys[v] = jnp.where(keep, keys[v], pk)
            idxs[v] = jnp.where(keep, idxs[v], pi)


def _seltop_kernel(mem_ref, wp_ref, bp_ref, idx_ref,
                   scr_a, scr_b, kscr, iscr):
    b = pl.program_id(0)
    blk = pl.program_id(1)

    @pl.when(b < B)
    def _scores():
        mem = mem_ref[0]
        logits = jnp.dot(mem, wp_ref[...],
                         preferred_element_type=jnp.float32)
        logits = logits + bp_ref[...]
        scores = jnp.max(jnp.transpose(logits), axis=0, keepdims=True)
        tok = blk * TBLK + lax.broadcasted_iota(jnp.int32, (1, TBLK), 1)
        x = tok & (W - 1)
        y = (tok >> 7) & (H - 1)
        valid = (x > 0) & (x < W - 1) & (y > 0) & (y < H - 1)
        sval = jnp.where(valid, scores, -jnp.inf)

        @pl.when(b % 2 == 0)
        def _():
            scr_a[pl.ds(blk, 1), :] = sval

        @pl.when(b % 2 == 1)
        def _():
            scr_b[pl.ds(blk, 1), :] = sval

    @pl.when(b > 0)
    def _sort():
        sub = lax.broadcasted_iota(jnp.int32, (8, 128), 0)
        lane = lax.broadcasted_iota(jnp.int32, (8, 128), 1)
        pos = sub * 128 + lane
        bitsets = [((pos >> jj) & 1) == 1 for jj in range(10)]
        for g in range(NGRP):
            @pl.when(blk == g)
            def _grp(g=g):
                sl = lambda r, v: r[:, v * 128:(v + 1) * 128]
                if g == 0:
                    odd = (b % 2) == 1
                    keys = [jnp.where(odd, sl(scr_a, v), sl(scr_b, v))
                            for v in range(NV)]
                    idxs = [(b - 1) * N + sub * TBLK + v * 128 + lane
                            for v in range(NV)]
                else:
                    keys = [sl(kscr, v) for v in range(NV)]
                    idxs = [sl(iscr, v) for v in range(NV)]
                _run_sort_ops(_SORT_GROUPS[g], keys, idxs, pos, bitsets)
                if g < NGRP - 1:
                    for v in range(NV):
                        kscr[:, v * 128:(v + 1) * 128] = keys[v]
                        iscr[:, v * 128:(v + 1) * 128] = idxs[v]
                else:
                    idx_ref[...] = idxs[_SORT_ROOT].reshape(1, 8, 128)


def _seltop_call(memory, wp, bp_t):
    nblk = N // TBLK
    return pl.pallas_call(
        _seltop_kernel,
        grid=(B + 1, nblk),
        in_specs=[
            pl.BlockSpec(
                (1, TBLK, C),
                lambda b, i: (jnp.minimum(b, B - 1),
                              jnp.where(b < B, i, nblk - 1), 0)),
            pl.BlockSpec((C, CPAD), lambda b, i: (0, 0)),
            pl.BlockSpec((1, CPAD), lambda b, i: (0, 0)),
        ],
        out_specs=pl.BlockSpec((1, 8, 128),
                               lambda b, i: (jnp.maximum(b, 1) - 1, 0, 0)),
        out_shape=jax.ShapeDtypeStruct((B, 8, 128), jnp.int32),
        scratch_shapes=[
            pltpu.VMEM((nblk, TBLK), jnp.float32),
            pltpu.VMEM((nblk, TBLK), jnp.float32),
            pltpu.VMEM((8, NV * 128), jnp.float32),
            pltpu.VMEM((8, NV * 128), jnp.int32),
        ],
    )(memory, wp, bp_t)


def _sc_gather(table, idx):
    info = plsc.get_sparse_core_info()
    ncores, nsub = info.num_cores, info.num_subcores
    nw = ncores * nsub
    nrows = idx.shape[0] * idx.shape[1]
    b_per_w = (nrows * idx.shape[2]) // nw
    mesh = plsc.VectorSubcoreMesh(core_axis_name="c", subcore_axis_name="s")

    @functools.partial(
        pl.kernel, mesh=mesh,
        out_type=jax.ShapeDtypeStruct((nw * b_per_w, C), jnp.float32),
        scratch_types=[
            pltpu.VMEM((b_per_w,), jnp.int32),
            pltpu.VMEM((b_per_w, C), jnp.float32),
            pltpu.SemaphoreType.DMA,
        ],
    )
    def gather_k(table_hbm, idx_hbm, out_hbm, idx_v, rows_v, sem):
        wid = lax.axis_index("s") * ncores + lax.axis_index("c")
        nr = idx_hbm.shape[1]
        pltpu.sync_copy(idx_hbm.at[wid // nr, wid % nr], idx_v)
        pltpu.async_copy(table_hbm.at[idx_v], rows_v, sem).wait()
        pltpu.sync_copy(rows_v, out_hbm.at[pl.ds(wid * b_per_w, b_per_w)])

    return gather_k(table, idx)


def _mlp_kernel(f_ref, i_ref, w1_ref, b1_ref, w2_ref, b2_ref,
                w3_ref, b3_ref, wc_ref, bc_ref, out_ref):
    x = f_ref[...]
    h = jnp.maximum(jnp.dot(x, w1_ref[...],
                            preferred_element_type=jnp.float32)
                    + b1_ref[...], 0.0)
    h = jnp.maximum(jnp.dot(h, w2_ref[...],
                            preferred_element_type=jnp.float32)
                    + b2_ref[...], 0.0)
    delta = jnp.dot(h, w3_ref[...],
                    preferred_element_type=jnp.float32) + b3_ref[...]
    lg = jnp.dot(x, wc_ref[...],
                 preferred_element_type=jnp.float32) + bc_ref[...]

    t = i_ref[...]
    xq = (t & (W - 1)).astype(jnp.float32)
    yq = ((t >> 7) & (H - 1)).astype(jnp.float32)
    cx = (xq + 0.5) * (1.0 / W)
    cy = (yq + 0.5) * (1.0 / H)
    dx = delta[:, 0:1]
    dy = delta[:, 1:2]
    dw = jnp.clip(delta[:, 2:3], -MAX_RATIO, MAX_RATIO)
    dh = jnp.clip(delta[:, 3:4], -MAX_RATIO, MAX_RATIO)
    gx = cx + 0.05 * dx
    gy = cy + 0.05 * dy
    gw = 0.05 * jnp.exp(dw)
    gh = 0.05 * jnp.exp(dh)
    x1 = jnp.clip(gx - 0.5 * gw, 0.0, 1.0)
    y1 = jnp.clip(gy - 0.5 * gh, 0.0, 1.0)
    x2 = jnp.clip(gx + 0.5 * gw, 0.0, 1.0)
    y2 = jnp.clip(gy + 0.5 * gh, 0.0, 1.0)
    out = jnp.concatenate(
        [x[:K], x1[:K], y1[:K], x2[:K], y2[:K], lg[:K, :NUM_CLASSES]],
        axis=1)
    out_ref[...] = out[None]


def _mlp_call(feats, idx_col, w1, b1, w2, b2, w3p, b3p, wcp, bcp):
    full = lambda r, c: pl.BlockSpec((r, c), lambda i: (0, 0))
    return pl.pallas_call(
        _mlp_kernel,
        grid=(B,),
        in_specs=[
            pl.BlockSpec((KPAD, C), lambda i: (i, 0)),
            pl.BlockSpec((KPAD, 1), lambda i: (i, 0)),
            full(C, C), full(1, C), full(C, C), full(1, C),
            full(C, 4), full(1, 4), full(C, CPAD), full(1, CPAD),
        ],
        out_specs=pl.BlockSpec((1, K, C + 4 + NUM_CLASSES),
                               lambda i: (i, 0, 0)),
        out_shape=jax.ShapeDtypeStruct((B, K, C + 4 + NUM_CLASSES),
                                       jnp.float32),
    )(feats, idx_col, w1, b1, w2, b2, w3p, b3p, wcp, bcp)


def kernel(memory, W_cls, b_cls, W1, b1, W2, b2, W3, b3):
    f32 = jnp.float32
    pad_c = CPAD - NUM_CLASSES
    wcp = jnp.pad(W_cls, ((0, 0), (0, pad_c)))
    bp_t = jnp.concatenate(
        [b_cls, jnp.full((pad_c,), -jnp.inf, f32)]).reshape(1, CPAD)
    b1r = b1.reshape(1, C)
    b2r = b2.reshape(1, C)
    b3r = b3.reshape(1, 4)

    gidx3 = _seltop_call(memory, wcp, bp_t)
    feats = _sc_gather(memory.reshape(B * N, C), gidx3)
    return _mlp_call(
        feats, gidx3.reshape(B * KPAD, 1),
        W1, b1r, W2, b2r, W3, b3r, wcp, bp_t)


# speedup vs baseline: 1.4504x; 1.1353x over previous
import functools

import numpy as np
import jax
import jax.numpy as jnp
from jax import lax
from jax.experimental import pallas as pl
from jax.experimental.pallas import tpu as pltpu
from jax.experimental.pallas import tpu_sc as plsc

B = 4
H = 128
W = 128
N = H * W
C = 256
NUM_CLASSES = 91
K = 900
KPAD = 1024
CPAD = 128
WH_RATIO_CLIP = 0.016
MAX_RATIO = float(np.abs(np.log(WH_RATIO_CLIP)))

TBLK = 4096
SROW = 2048

NV = 16
VSZ = 1024


def _beats(ka, ia, kb, ib):
    return (ka > kb) | ((ka == kb) & (ia < ib))


NGRP = N // TBLK


def _sort_schedule():
    ops = []
    for m in range(1, 11):
        k = 1 << m
        for jj in range(m - 1, -1, -1):
            spec = ('elem', k) if k < VSZ else ('uni',)
            ops.append((('cx', jj, spec,
                         [(v, v % 2 == 1) for v in range(NV)]), NV))
    alive = list(range(NV))
    while len(alive) > 1:
        nxt = []
        for t in range(len(alive) // 2):
            a, c = alive[2 * t], alive[2 * t + 1]
            ops.append((('win', a, c, None), 1))
            nxt.append(a)
        for jj in range(9, -1, -1):
            ops.append((('cx', jj, ('uni',),
                         [(v, t % 2 == 1) for t, v in enumerate(nxt)]),
                        len(nxt)))
        alive = nxt
    total = sum(w for _, w in ops)
    groups, acc, gi = [[] for _ in range(NGRP)], 0, 0
    for op, w in ops:
        if acc >= (gi + 1) * total / NGRP and gi < NGRP - 1:
            gi += 1
        groups[gi].append(op)
        acc += w
    return groups, alive[0]


_SORT_GROUPS, _SORT_ROOT = _sort_schedule()


def _run_sort_ops(opsg, keys, idxs, pos, bitsets):
    for op in opsg:
        if op[0] == 'win':
            _, a, c, _ = op
            aw = _beats(keys[a], idxs[a], keys[c], idxs[c])
            keys[a] = jnp.where(aw, keys[a], keys[c])
            idxs[a] = jnp.where(aw, idxs[a], idxs[c])
            continue
        _, jj, spec, pairs = op
        j = 1 << jj
        axis, shift, size = (1, j, 128) if j < 128 else (0, j >> 7, 8)
        bs = bitsets[jj]
        if spec[0] == 'elem':
            base = ~(bs ^ ((pos & spec[1]) != 0))
        else:
            base = ~bs
        nbase = ~base
        for v, inv in pairs:
            hw = nbase if inv else base
            pk = jnp.where(bs, pltpu.roll(keys[v], shift, axis),
                           pltpu.roll(keys[v], size - shift, axis))
            pi = jnp.where(bs, pltpu.roll(idxs[v], shift, axis),
                           pltpu.roll(idxs[v], size - shift, axis))
            sw = _beats(keys[v], idxs[v], pk, pi)
            keep = ~(sw ^ hw)
            keys[v] = jnp.where(keep, keys[v], pk)
            idxs[v] = jnp.where(keep, idxs[v], pi)


def _seltop_kernel(mem_ref, wp_ref, bp_ref, idx_ref,
                   scr_a, scr_b, kscr, iscr):
    b = pl.program_id(0)
    blk = pl.program_id(1)

    @pl.when(b < B)
    def _scores():
        mem = mem_ref[0]
        logits = jnp.dot(mem, wp_ref[...],
                         preferred_element_type=jnp.float32)
        logits = logits + bp_ref[...]
        scores = jnp.max(jnp.transpose(logits), axis=0, keepdims=True)
        tok = blk * TBLK + lax.broadcasted_iota(jnp.int32, (1, TBLK), 1)
        x = tok & (W - 1)
        y = (tok >> 7) & (H - 1)
        valid = (x > 0) & (x < W - 1) & (y > 0) & (y < H - 1)
        sval = jnp.where(valid, scores, -jnp.inf)
        nsub = TBLK // SROW

        @pl.when(b % 2 == 0)
        def _():
            for c2 in range(nsub):
                scr_a[pl.ds(blk * nsub + c2, 1), :] = \
                    sval[:, c2 * SROW:(c2 + 1) * SROW]

        @pl.when(b % 2 == 1)
        def _():
            for c2 in range(nsub):
                scr_b[pl.ds(blk * nsub + c2, 1), :] = \
                    sval[:, c2 * SROW:(c2 + 1) * SROW]

    @pl.when(b > 0)
    def _sort():
        sub = lax.broadcasted_iota(jnp.int32, (8, 128), 0)
        lane = lax.broadcasted_iota(jnp.int32, (8, 128), 1)
        pos = sub * 128 + lane
        bitsets = [((pos >> jj) & 1) == 1 for jj in range(10)]
        for g in range(NGRP):
            @pl.when(blk == g)
            def _grp(g=g):
                sl = lambda r, v: r[:, v * 128:(v + 1) * 128]
                if g == 0:
                    odd = (b % 2) == 1
                    keys = [jnp.where(odd, sl(scr_a, v), sl(scr_b, v))
                            for v in range(NV)]
                    idxs = [(b - 1) * N + sub * SROW + v * 128 + lane
                            for v in range(NV)]
                else:
                    keys = [sl(kscr, v) for v in range(NV)]
                    idxs = [sl(iscr, v) for v in range(NV)]
                _run_sort_ops(_SORT_GROUPS[g], keys, idxs, pos, bitsets)
                if g < NGRP - 1:
                    for v in range(NV):
                        kscr[:, v * 128:(v + 1) * 128] = keys[v]
                        iscr[:, v * 128:(v + 1) * 128] = idxs[v]
                else:
                    idx_ref[...] = idxs[_SORT_ROOT].reshape(1, 8, 128)


def _seltop_call(memory, wp, bp_t):
    nblk = N // TBLK
    return pl.pallas_call(
        _seltop_kernel,
        grid=(B + 1, nblk),
        in_specs=[
            pl.BlockSpec(
                (1, TBLK, C),
                lambda b, i: (jnp.minimum(b, B - 1),
                              jnp.where(b < B, i, nblk - 1), 0)),
            pl.BlockSpec((C, CPAD), lambda b, i: (0, 0)),
            pl.BlockSpec((1, CPAD), lambda b, i: (0, 0)),
        ],
        out_specs=pl.BlockSpec((1, 8, 128),
                               lambda b, i: (jnp.maximum(b, 1) - 1, 0, 0)),
        out_shape=jax.ShapeDtypeStruct((B, 8, 128), jnp.int32),
        scratch_shapes=[
            pltpu.VMEM((N // SROW, SROW), jnp.float32),
            pltpu.VMEM((N // SROW, SROW), jnp.float32),
            pltpu.VMEM((8, NV * 128), jnp.float32),
            pltpu.VMEM((8, NV * 128), jnp.int32),
        ],
    )(memory, wp, bp_t)


def _sc_gather(table, idx):
    info = plsc.get_sparse_core_info()
    ncores, nsub = info.num_cores, info.num_subcores
    nw = ncores * nsub
    nrows = idx.shape[0] * idx.shape[1]
    b_per_w = (nrows * idx.shape[2]) // nw
    mesh = plsc.VectorSubcoreMesh(core_axis_name="c", subcore_axis_name="s")

    @functools.partial(
        pl.kernel, mesh=mesh,
        out_type=jax.ShapeDtypeStruct((nw * b_per_w, C), jnp.float32),
        scratch_types=[
            pltpu.VMEM((b_per_w,), jnp.int32),
            pltpu.VMEM((b_per_w, C), jnp.float32),
            pltpu.SemaphoreType.DMA,
        ],
    )
    def gather_k(table_hbm, idx_hbm, out_hbm, idx_v, rows_v, sem):
        wid = lax.axis_index("s") * ncores + lax.axis_index("c")
        nr = idx_hbm.shape[1]
        pltpu.sync_copy(idx_hbm.at[wid // nr, wid % nr], idx_v)
        pltpu.async_copy(table_hbm.at[idx_v], rows_v, sem).wait()
        pltpu.sync_copy(rows_v, out_hbm.at[pl.ds(wid * b_per_w, b_per_w)])

    return gather_k(table, idx)


def _mlp_kernel(f_ref, i_ref, w1_ref, b1_ref, w2_ref, b2_ref,
                w3_ref, b3_ref, wc_ref, bc_ref, out_ref):
    x = f_ref[...]
    h = jnp.maximum(jnp.dot(x, w1_ref[...],
                            preferred_element_type=jnp.float32)
                    + b1_ref[...], 0.0)
    h = jnp.maximum(jnp.dot(h, w2_ref[...],
                            preferred_element_type=jnp.float32)
                    + b2_ref[...], 0.0)
    delta = jnp.dot(h, w3_ref[...],
                    preferred_element_type=jnp.float32) + b3_ref[...]
    lg = jnp.dot(x, wc_ref[...],
                 preferred_element_type=jnp.float32) + bc_ref[...]

    t = i_ref[...]
    xq = (t & (W - 1)).astype(jnp.float32)
    yq = ((t >> 7) & (H - 1)).astype(jnp.float32)
    cx = (xq + 0.5) * (1.0 / W)
    cy = (yq + 0.5) * (1.0 / H)
    dx = delta[:, 0:1]
    dy = delta[:, 1:2]
    dw = jnp.clip(delta[:, 2:3], -MAX_RATIO, MAX_RATIO)
    dh = jnp.clip(delta[:, 3:4], -MAX_RATIO, MAX_RATIO)
    gx = cx + 0.05 * dx
    gy = cy + 0.05 * dy
    gw = 0.05 * jnp.exp(dw)
    gh = 0.05 * jnp.exp(dh)
    x1 = jnp.clip(gx - 0.5 * gw, 0.0, 1.0)
    y1 = jnp.clip(gy - 0.5 * gh, 0.0, 1.0)
    x2 = jnp.clip(gx + 0.5 * gw, 0.0, 1.0)
    y2 = jnp.clip(gy + 0.5 * gh, 0.0, 1.0)
    out = jnp.concatenate(
        [x[:K], x1[:K], y1[:K], x2[:K], y2[:K], lg[:K, :NUM_CLASSES]],
        axis=1)
    out_ref[...] = out[None]


def _mlp_call(feats, idx_col, w1, b1, w2, b2, w3p, b3p, wcp, bcp):
    full = lambda r, c: pl.BlockSpec((r, c), lambda i: (0, 0))
    return pl.pallas_call(
        _mlp_kernel,
        grid=(B,),
        in_specs=[
            pl.BlockSpec((KPAD, C), lambda i: (i, 0)),
            pl.BlockSpec((KPAD, 1), lambda i: (i, 0)),
            full(C, C), full(1, C), full(C, C), full(1, C),
            full(C, 4), full(1, 4), full(C, CPAD), full(1, CPAD),
        ],
        out_specs=pl.BlockSpec((1, K, C + 4 + NUM_CLASSES),
                               lambda i: (i, 0, 0)),
        out_shape=jax.ShapeDtypeStruct((B, K, C + 4 + NUM_CLASSES),
                                       jnp.float32),
    )(feats, idx_col, w1, b1, w2, b2, w3p, b3p, wcp, bcp)


def kernel(memory, W_cls, b_cls, W1, b1, W2, b2, W3, b3):
    f32 = jnp.float32
    pad_c = CPAD - NUM_CLASSES
    wcp = jnp.pad(W_cls, ((0, 0), (0, pad_c)))
    bp_t = jnp.concatenate(
        [b_cls, jnp.full((pad_c,), -jnp.inf, f32)]).reshape(1, CPAD)
    b1r = b1.reshape(1, C)
    b2r = b2.reshape(1, C)
    b3r = b3.reshape(1, 4)

    gidx3 = _seltop_call(memory, wcp, bp_t)
    feats = _sc_gather(memory.reshape(B * N, C), gidx3)
    return _mlp_call(
        feats, gidx3.reshape(B * KPAD, 1),
        W1, b1r, W2, b2r, W3, b3r, wcp, bp_t)


# speedup vs baseline: 1.5146x; 1.0442x over previous
import functools

import numpy as np
import jax
import jax.numpy as jnp
from jax import lax
from jax.experimental import pallas as pl
from jax.experimental.pallas import tpu as pltpu
from jax.experimental.pallas import tpu_sc as plsc

B = 4
H = 128
W = 128
N = H * W
C = 256
NUM_CLASSES = 91
K = 900
KPAD = 1024
CPAD = 128
WH_RATIO_CLIP = 0.016
MAX_RATIO = float(np.abs(np.log(WH_RATIO_CLIP)))

TBLK = 8192
SROW = 2048

NV = 16
VSZ = 1024


def _beats(ka, ia, kb, ib):
    return (ka > kb) | ((ka == kb) & (ia < ib))


NGRP = N // TBLK


def _sort_schedule():
    ops = []
    for m in range(1, 11):
        k = 1 << m
        for jj in range(m - 1, -1, -1):
            spec = ('elem', k) if k < VSZ else ('uni',)
            ops.append((('cx', jj, spec,
                         [(v, v % 2 == 1) for v in range(NV)]), NV))
    alive = list(range(NV))
    while len(alive) > 1:
        nxt = []
        for t in range(len(alive) // 2):
            a, c = alive[2 * t], alive[2 * t + 1]
            ops.append((('win', a, c, None), 1))
            nxt.append(a)
        for jj in range(9, -1, -1):
            ops.append((('cx', jj, ('uni',),
                         [(v, t % 2 == 1) for t, v in enumerate(nxt)]),
                        len(nxt)))
        alive = nxt
    total = sum(w for _, w in ops)
    groups, acc, gi = [[] for _ in range(NGRP)], 0, 0
    for op, w in ops:
        if acc >= (gi + 1) * total / NGRP and gi < NGRP - 1:
            gi += 1
        groups[gi].append(op)
        acc += w
    return groups, alive[0]


_SORT_GROUPS, _SORT_ROOT = _sort_schedule()


def _run_sort_ops(opsg, keys, idxs, pos, bitsets):
    for op in opsg:
        if op[0] == 'win':
            _, a, c, _ = op
            aw = _beats(keys[a], idxs[a], keys[c], idxs[c])
            keys[a] = jnp.where(aw, keys[a], keys[c])
            idxs[a] = jnp.where(aw, idxs[a], idxs[c])
            continue
        _, jj, spec, pairs = op
        j = 1 << jj
        axis, shift, size = (1, j, 128) if j < 128 else (0, j >> 7, 8)
        bs = bitsets[jj]
        if spec[0] == 'elem':
            base = ~(bs ^ ((pos & spec[1]) != 0))
        else:
            base = ~bs
        nbase = ~base
        for v, inv in pairs:
            hw = nbase if inv else base
            pk = jnp.where(bs, pltpu.roll(keys[v], shift, axis),
                           pltpu.roll(keys[v], size - shift, axis))
            pi = jnp.where(bs, pltpu.roll(idxs[v], shift, axis),
                           pltpu.roll(idxs[v], size - shift, axis))
            sw = _beats(keys[v], idxs[v], pk, pi)
            keep = ~(sw ^ hw)
            keys[v] = jnp.where(keep, keys[v], pk)
            idxs[v] = jnp.where(keep, idxs[v], pi)


def _seltop_kernel(mem_ref, wp_ref, bp_ref, idx_ref,
                   scr_a, scr_b, kscr, iscr):
    b = pl.program_id(0)
    blk = pl.program_id(1)

    @pl.when(b < B)
    def _scores():
        mem = mem_ref[0]
        logits = jnp.dot(mem, wp_ref[...],
                         preferred_element_type=jnp.float32)
        logits = logits + bp_ref[...]
        scores = jnp.max(jnp.transpose(logits), axis=0, keepdims=True)
        tok = blk * TBLK + lax.broadcasted_iota(jnp.int32, (1, TBLK), 1)
        x = tok & (W - 1)
        y = (tok >> 7) & (H - 1)
        valid = (x > 0) & (x < W - 1) & (y > 0) & (y < H - 1)
        sval = jnp.where(valid, scores, -jnp.inf)
        nsub = TBLK // SROW

        @pl.when(b % 2 == 0)
        def _():
            for c2 in range(nsub):
                scr_a[pl.ds(blk * nsub + c2, 1), :] = \
                    sval[:, c2 * SROW:(c2 + 1) * SROW]

        @pl.when(b % 2 == 1)
        def _():
            for c2 in range(nsub):
                scr_b[pl.ds(blk * nsub + c2, 1), :] = \
                    sval[:, c2 * SROW:(c2 + 1) * SROW]

    @pl.when(b > 0)
    def _sort():
        sub = lax.broadcasted_iota(jnp.int32, (8, 128), 0)
        lane = lax.broadcasted_iota(jnp.int32, (8, 128), 1)
        pos = sub * 128 + lane
        bitsets = [((pos >> jj) & 1) == 1 for jj in range(10)]
        for g in range(NGRP):
            @pl.when(blk == g)
            def _grp(g=g):
                sl = lambda r, v: r[:, v * 128:(v + 1) * 128]
                if g == 0:
                    odd = (b % 2) == 1
                    keys = [jnp.where(odd, sl(scr_a, v), sl(scr_b, v))
                            for v in range(NV)]
                    idxs = [(b - 1) * N + sub * SROW + v * 128 + lane
                            for v in range(NV)]
                else:
                    keys = [sl(kscr, v) for v in range(NV)]
                    idxs = [sl(iscr, v) for v in range(NV)]
                _run_sort_ops(_SORT_GROUPS[g], keys, idxs, pos, bitsets)
                if g < NGRP - 1:
                    for v in range(NV):
                        kscr[:, v * 128:(v + 1) * 128] = keys[v]
                        iscr[:, v * 128:(v + 1) * 128] = idxs[v]
                else:
                    idx_ref[...] = idxs[_SORT_ROOT].reshape(1, 8, 128)


def _seltop_call(memory, wp, bp_t):
    nblk = N // TBLK
    return pl.pallas_call(
        _seltop_kernel,
        grid=(B + 1, nblk),
        in_specs=[
            pl.BlockSpec(
                (1, TBLK, C),
                lambda b, i: (jnp.minimum(b, B - 1),
                              jnp.where(b < B, i, nblk - 1), 0)),
            pl.BlockSpec((C, CPAD), lambda b, i: (0, 0)),
            pl.BlockSpec((1, CPAD), lambda b, i: (0, 0)),
        ],
        out_specs=pl.BlockSpec((1, 8, 128),
                               lambda b, i: (jnp.maximum(b, 1) - 1, 0, 0)),
        out_shape=jax.ShapeDtypeStruct((B, 8, 128), jnp.int32),
        scratch_shapes=[
            pltpu.VMEM((N // SROW, SROW), jnp.float32),
            pltpu.VMEM((N // SROW, SROW), jnp.float32),
            pltpu.VMEM((8, NV * 128), jnp.float32),
            pltpu.VMEM((8, NV * 128), jnp.int32),
        ],
    )(memory, wp, bp_t)


def _sc_gather(table, idx):
    info = plsc.get_sparse_core_info()
    ncores, nsub = info.num_cores, info.num_subcores
    nw = ncores * nsub
    nrows = idx.shape[0] * idx.shape[1]
    b_per_w = (nrows * idx.shape[2]) // nw
    mesh = plsc.VectorSubcoreMesh(core_axis_name="c", subcore_axis_name="s")

    @functools.partial(
        pl.kernel, mesh=mesh,
        out_type=jax.ShapeDtypeStruct((nw * b_per_w, C), jnp.float32),
        scratch_types=[
            pltpu.VMEM((b_per_w,), jnp.int32),
            pltpu.VMEM((b_per_w, C), jnp.float32),
            pltpu.SemaphoreType.DMA,
        ],
    )
    def gather_k(table_hbm, idx_hbm, out_hbm, idx_v, rows_v, sem):
        wid = lax.axis_index("s") * ncores + lax.axis_index("c")
        nr = idx_hbm.shape[1]
        pltpu.sync_copy(idx_hbm.at[wid // nr, wid % nr], idx_v)
        pltpu.async_copy(table_hbm.at[idx_v], rows_v, sem).wait()
        pltpu.sync_copy(rows_v, out_hbm.at[pl.ds(wid * b_per_w, b_per_w)])

    return gather_k(table, idx)


def _mlp_kernel(f_ref, i_ref, w1_ref, b1_ref, w2_ref, b2_ref,
                w3_ref, b3_ref, wc_ref, bc_ref, out_ref):
    x = f_ref[...]
    h = jnp.maximum(jnp.dot(x, w1_ref[...],
                            preferred_element_type=jnp.float32)
                    + b1_ref[...], 0.0)
    h = jnp.maximum(jnp.dot(h, w2_ref[...],
                            preferred_element_type=jnp.float32)
                    + b2_ref[...], 0.0)
    delta = jnp.dot(h, w3_ref[...],
                    preferred_element_type=jnp.float32) + b3_ref[...]
    lg = jnp.dot(x, wc_ref[...],
                 preferred_element_type=jnp.float32) + bc_ref[...]

    t = i_ref[...]
    xq = (t & (W - 1)).astype(jnp.float32)
    yq = ((t >> 7) & (H - 1)).astype(jnp.float32)
    cx = (xq + 0.5) * (1.0 / W)
    cy = (yq + 0.5) * (1.0 / H)
    dx = delta[:, 0:1]
    dy = delta[:, 1:2]
    dw = jnp.clip(delta[:, 2:3], -MAX_RATIO, MAX_RATIO)
    dh = jnp.clip(delta[:, 3:4], -MAX_RATIO, MAX_RATIO)
    gx = cx + 0.05 * dx
    gy = cy + 0.05 * dy
    gw = 0.05 * jnp.exp(dw)
    gh = 0.05 * jnp.exp(dh)
    x1 = jnp.clip(gx - 0.5 * gw, 0.0, 1.0)
    y1 = jnp.clip(gy - 0.5 * gh, 0.0, 1.0)
    x2 = jnp.clip(gx + 0.5 * gw, 0.0, 1.0)
    y2 = jnp.clip(gy + 0.5 * gh, 0.0, 1.0)
    out = jnp.concatenate(
        [x[:K], x1[:K], y1[:K], x2[:K], y2[:K], lg[:K, :NUM_CLASSES]],
        axis=1)
    out_ref[...] = out[None]


def _mlp_call(feats, idx_col, w1, b1, w2, b2, w3p, b3p, wcp, bcp):
    full = lambda r, c: pl.BlockSpec((r, c), lambda i: (0, 0))
    return pl.pallas_call(
        _mlp_kernel,
        grid=(B,),
        in_specs=[
            pl.BlockSpec((KPAD, C), lambda i: (i, 0)),
            pl.BlockSpec((KPAD, 1), lambda i: (i, 0)),
            full(C, C), full(1, C), full(C, C), full(1, C),
            full(C, 4), full(1, 4), full(C, CPAD), full(1, CPAD),
        ],
        out_specs=pl.BlockSpec((1, K, C + 4 + NUM_CLASSES),
                               lambda i: (i, 0, 0)),
        out_shape=jax.ShapeDtypeStruct((B, K, C + 4 + NUM_CLASSES),
                                       jnp.float32),
    )(feats, idx_col, w1, b1, w2, b2, w3p, b3p, wcp, bcp)


def kernel(memory, W_cls, b_cls, W1, b1, W2, b2, W3, b3):
    f32 = jnp.float32
    pad_c = CPAD - NUM_CLASSES
    wcp = jnp.pad(W_cls, ((0, 0), (0, pad_c)))
    bp_t = jnp.concatenate(
        [b_cls, jnp.full((pad_c,), -jnp.inf, f32)]).reshape(1, CPAD)
    b1r = b1.reshape(1, C)
    b2r = b2.reshape(1, C)
    b3r = b3.reshape(1, 4)

    gidx3 = _seltop_call(memory, wcp, bp_t)
    feats = _sc_gather(memory.reshape(B * N, C), gidx3)
    return _mlp_call(
        feats, gidx3.reshape(B * KPAD, 1),
        W1, b1r, W2, b2r, W3, b3r, wcp, bp_t)
